# baseline (speedup 1.0000x reference)
import functools

import numpy as np
import jax
import jax.numpy as jnp
from jax import lax
from jax.experimental import pallas as pl
from jax.experimental.pallas import tpu as pltpu
from jax.experimental.pallas import tpu_sc as plsc

D = 128
E = 256
S = 64
BUCKET = 64
N_BLOCK = 2


def _dot(a, b, dims):
    return jax.lax.dot_general(a, b, (dims, ((), ())),
                               preferred_element_type=jnp.float32)


def _dot_fast(a, b, dims):
    return jax.lax.dot_general(a.astype(jnp.bfloat16), b.astype(jnp.bfloat16),
                               (dims, ((), ())),
                               preferred_element_type=jnp.float32)


def _proj_body(xs_ref, w_ref, b_ref, o_ref):
    o_ref[0] = _dot(xs_ref[0], w_ref[...], ((1,), (0,))) + b_ref[...]


def _proj(xs, W_in, b_in):
    Bn, n0, din = xs.shape
    RB = 2048
    return pl.pallas_call(
        _proj_body,
        grid=(Bn, n0 // RB),
        in_specs=[
            pl.BlockSpec((1, RB, din), lambda s, r: (s, r, 0)),
            pl.BlockSpec((din, D), lambda s, r: (0, 0)),
            pl.BlockSpec((1, D), lambda s, r: (0, 0)),
        ],
        out_specs=pl.BlockSpec((1, RB, D), lambda s, r: (s, r, 0)),
        out_shape=jax.ShapeDtypeStruct((Bn, n0, D), jnp.float32),
    )(xs, W_in, b_in.reshape(1, D))


def _cw_body(x_ref, cw_ref):
    x = x_ref[0]
    n = x.shape[0]
    ones = jnp.ones((1, D), jnp.float32)
    lens2 = _dot(ones, x * x, ((1,), (1,)))
    m = jnp.max(lens2)
    iota = jax.lax.broadcasted_iota(jnp.int32, (1, n), 1)
    idx = jnp.min(jnp.where(lens2 == m, iota, n))
    onehot = (iota == idx).astype(jnp.float32)
    v1 = _dot(onehot, x, ((1,), (0,)))
    v1n = jnp.sqrt(jnp.sum(v1 * v1))
    dots = _dot(v1, x, ((1,), (1,)))
    denom = jnp.maximum(v1n * jnp.sqrt(lens2), 1e-8)
    cw_ref[0] = dots / denom


def _cosine_weights(x):
    Bn, n, _ = x.shape
    out = pl.pallas_call(
        _cw_body,
        grid=(Bn,),
        in_specs=[pl.BlockSpec((1, n, D), lambda s: (s, 0, 0))],
        out_specs=pl.BlockSpec((1, 1, n), lambda s: (s, 0, 0)),
        out_shape=jax.ShapeDtypeStruct((Bn, 1, n), jnp.float32),
    )(x)
    return out[:, 0]


def _gau_body(G, g_ref, lng_ref, lnb_ref, wuv_ref, buv_ref, wz_ref, bz_ref,
              gam_ref, bet_ref, wo_ref, bo_ref, trg_ref, trb_ref, wtr_ref,
              btr_ref, gys_ref, ys_ref):
    x = g_ref[0]
    PAIR = G % 2 == 0
    if PAIR:
        r2 = jax.lax.broadcasted_iota(jnp.int32, (2 * BUCKET, 2 * BUCKET), 0)
        c2 = jax.lax.broadcasted_iota(jnp.int32, (2 * BUCKET, 2 * BUCKET), 1)
        bdmask = (r2 // BUCKET == c2 // BUCKET).astype(jnp.float32)
    onesd = jnp.full((D, 1), 1.0 / D, jnp.float32)
    for j in range(N_BLOCK):
        mean = _dot(x, onesd, ((1,), (0,)))
        xc = x - mean
        var = _dot(xc * xc, onesd, ((1,), (0,)))
        inv = jax.lax.rsqrt(var + 1e-5)
        xn = xc * inv * lng_ref[j:j + 1] + lnb_ref[j:j + 1]
        uv = _dot(xn.astype(jnp.bfloat16), wuv_ref[j], ((1,), (0,))) \
            + buv_ref[j:j + 1]
        uv = uv * jax.nn.sigmoid(uv)
        u = uv[:, :E]
        v = uv[:, E:]
        z = _dot(xn, wz_ref[j], ((1,), (0,))) + bz_ref[j:j + 1]
        q = z * gam_ref[2 * j:2 * j + 1] + bet_ref[2 * j:2 * j + 1]
        k = z * gam_ref[2 * j + 1:2 * j + 2] + bet_ref[2 * j + 1:2 * j + 2]
        outs = []
        if PAIR:
            for t in range(G // 2):
                sl = slice(t * 2 * BUCKET, (t + 1) * 2 * BUCKET)
                qk = _dot(q[sl], k[sl], ((1,), (1,))) * (1.0 / BUCKET)
                A = jnp.square(jnp.maximum(qk, 0.0)) * bdmask
                outs.append(_dot(A, v[sl], ((1,), (0,))))
        else:
            for t in range(G):
                sl = slice(t * BUCKET, (t + 1) * BUCKET)
                qk = _dot(q[sl], k[sl], ((1,), (1,))) * (1.0 / BUCKET)
                A = jnp.square(jnp.maximum(qk, 0.0))
                outs.append(_dot(A, v[sl], ((1,), (0,))))
        av = jnp.concatenate(outs, axis=0) if len(outs) > 1 else outs[0]
        x = x + _dot((u * av).astype(jnp.bfloat16), wo_ref[j], ((1,), (0,))) \
            + bo_ref[j:j + 1]
    rg = jax.lax.broadcasted_iota(jnp.int32, (G, G * BUCKET), 0)
    cg = jax.lax.broadcasted_iota(jnp.int32, (G, G * BUCKET), 1)
    pool = jnp.where(cg // BUCKET == rg, 1.0 / BUCKET, 0.0)
    rows = _dot(pool, x, ((1,), (0,)))
    gys_ref[0] = rows
    m2 = jnp.mean(rows, axis=1, keepdims=True)
    v2 = jnp.mean((rows - m2) ** 2, axis=1, keepdims=True)
    t_ = (rows - m2) / jnp.sqrt(v2 + 1e-5) * trg_ref[...] + trb_ref[...]
    t_ = jnp.where(t_ >= 0, t_, 0.01 * t_)
    ys_ref[0] = _dot(t_, wtr_ref[...], ((1,), (0,))) + btr_ref[...]


def _gau_level(g, G, wts):
    (lng, lnb, wuv, buv, wz, bz, gam, bet, wo, bo, trg, trb, wtr, btr) = wts
    Bn, rows, _ = g.shape
    nbp = rows // BUCKET
    grid = (Bn, nbp // G)
    const3 = lambda shp: pl.BlockSpec(shp, lambda s, b: (0, 0, 0))
    const2 = lambda shp: pl.BlockSpec(shp, lambda s, b: (0, 0))
    return pl.pallas_call(
        functools.partial(_gau_body, G),
        grid=grid,
        in_specs=[
            pl.BlockSpec((1, G * BUCKET, D), lambda s, b: (s, b, 0)),
            const2((N_BLOCK, D)), const2((N_BLOCK, D)),
            const3((N_BLOCK, D, 2 * E)), const2((N_BLOCK, 2 * E)),
            const3((N_BLOCK, D, S)), const2((N_BLOCK, S)),
            const2((2 * N_BLOCK, S)), const2((2 * N_BLOCK, S)),
            const3((N_BLOCK, E, D)), const2((N_BLOCK, D)),
            const2((1, D)), const2((1, D)),
            const2((D, D)), const2((1, D)),
        ],
        out_specs=[
            pl.BlockSpec((1, G, D), lambda s, b: (s, b, 0)),
            pl.BlockSpec((1, G, D), lambda s, b: (s, b, 0)),
        ],
        out_shape=[
            jax.ShapeDtypeStruct((Bn, nbp, D), jnp.float32),
            jax.ShapeDtypeStruct((Bn, nbp, D), jnp.float32),
        ],
    )(g, lng, lnb, wuv, buv, wz, bz, gam, bet, wo, bo, trg, trb, wtr, btr)


def _rank_desc(cw_row, valid_row, n):
    ii = jax.lax.broadcasted_iota(jnp.int32, (n, n), 0)
    jj = jax.lax.broadcasted_iota(jnp.int32, (n, n), 1)
    eye = (ii == jj).astype(jnp.float32)
    cw_col = _dot(eye, cw_row, ((1,), (1,)))
    gt = jnp.where((cw_row > cw_col) & valid_row, 1.0, 0.0)
    tie = jnp.where((cw_row == cw_col) & (jj < ii) & valid_row, 1.0, 0.0)
    ones = jnp.ones((n, 1), jnp.float32)
    return _dot(gt + tie, ones, ((1,), (0,)))


def _sort_pad(x, n_valid, n_rows, n_out):
    iota_r = jax.lax.broadcasted_iota(jnp.int32, (1, n_rows), 1)
    valid = iota_r < n_valid
    ones = jnp.ones((1, D), jnp.float32)
    lens2 = _dot(ones, x * x, ((1,), (1,)))
    lens2m = jnp.where(valid, lens2, -1.0)
    m = jnp.max(lens2m)
    idx = jnp.min(jnp.where(lens2m == m, iota_r, n_rows))
    onehot = (iota_r == idx).astype(jnp.float32)
    v1 = _dot(onehot, x, ((1,), (0,)))
    v1n = jnp.sqrt(jnp.sum(v1 * v1))
    dots = _dot(v1, x, ((1,), (1,)))
    denom = jnp.maximum(v1n * jnp.sqrt(jnp.maximum(lens2, 0.0)), 1e-8)
    cw = jnp.where(valid, dots / denom, -3.0)
    rank = _rank_desc(cw, valid, n_rows)
    validc = jax.lax.broadcasted_iota(jnp.int32, (n_rows, 1), 0) < n_valid
    ranks = jnp.where(validc, rank, -1.0)
    kk = jax.lax.broadcasted_iota(jnp.int32, (n_rows, n_out), 1)
    PT = jnp.where(ranks == kk.astype(jnp.float32), 1.0, 0.0)
    xs_sorted = _dot(PT, x, ((0,), (0,)))
    ko = jax.lax.broadcasted_iota(jnp.int32, (n_out, D), 0)
    do = jax.lax.broadcasted_iota(jnp.int32, (n_out, D), 1)
    padfill = jnp.where((ko >= n_valid) & (do == (ko - n_valid) % D),
                        1.0, 0.0)
    return xs_sorted + padfill


def _gau_blocks(x, G, wts, bdmask):
    (lng, lnb, wuv, buv, wz, bz, gam, bet, wo, bo) = wts
    for j in range(N_BLOCK):
        mean = jnp.mean(x, axis=1, keepdims=True)
        var = jnp.mean((x - mean) ** 2, axis=1, keepdims=True)
        xn = (x - mean) / jnp.sqrt(var + 1e-5) * lng[j:j + 1] + lnb[j:j + 1]
        uv = _dot(xn.astype(jnp.bfloat16), wuv[j], ((1,), (0,))) + buv[j:j + 1]
        uv = uv * jax.nn.sigmoid(uv)
        u = uv[:, :E]
        v = uv[:, E:]
        z = _dot(xn, wz[j], ((1,), (0,))) + bz[j:j + 1]
        q = z * gam[2 * j:2 * j + 1] + bet[2 * j:2 * j + 1]
        k = z * gam[2 * j + 1:2 * j + 2] + bet[2 * j + 1:2 * j + 2]
        qk = _dot(q, k, ((1,), (1,))) * (1.0 / BUCKET)
        A = jnp.square(jnp.maximum(qk, 0.0))
        if bdmask is not None:
            A = A * bdmask
        av = _dot(A, v, ((1,), (0,)))
        x = x + _dot((u * av).astype(jnp.bfloat16), wo[j], ((1,), (0,))) \
            + bo[j:j + 1]
    return x


def _trans_head(rows, trg, trb, wtr, btr):
    m = jnp.mean(rows, axis=1, keepdims=True)
    v = jnp.mean((rows - m) ** 2, axis=1, keepdims=True)
    t_ = (rows - m) / jnp.sqrt(v + 1e-5) * trg + trb
    t_ = jnp.where(t_ >= 0, t_, 0.01 * t_)
    return _dot(t_, wtr, ((1,), (0,))) + btr


def _tail_body(n1, gw, g_ref, ys0_ref, lng_ref, lnb_ref, wuv_ref, buv_ref,
               wz_ref, bz_ref, gam_ref, bet_ref, wo_ref, bo_ref, trg_ref,
               trb_ref, wtr_ref, btr_ref, og_ref, ob_ref, wout_ref, bout_ref,
               o_ref):
    x = g_ref[0]
    n_rows = x.shape[0]
    wts = (lng_ref[...], lnb_ref[...], wuv_ref[...], buv_ref[...],
           wz_ref[...], bz_ref[...], gam_ref[...], bet_ref[...],
           wo_ref[...], bo_ref[...])
    trg, trb = trg_ref[...], trb_ref[...]
    wtr, btr = wtr_ref[...], btr_ref[...]

    g1 = _sort_pad(x, n1, n_rows, 5 * BUCKET)
    r5 = jax.lax.broadcasted_iota(jnp.int32, (5 * BUCKET, 5 * BUCKET), 0)
    c5 = jax.lax.broadcasted_iota(jnp.int32, (5 * BUCKET, 5 * BUCKET), 1)
    bdmask = (r5 // BUCKET == c5 // BUCKET).astype(jnp.float32)
    g1 = _gau_blocks(g1, 5, wts, bdmask)
    rg = jax.lax.broadcasted_iota(jnp.int32, (5, 5 * BUCKET), 0)
    cg = jax.lax.broadcasted_iota(jnp.int32, (5, 5 * BUCKET), 1)
    pool1 = jnp.where(cg // BUCKET == rg, 1.0 / BUCKET, 0.0)
    rows5 = _dot(pool1, g1, ((1,), (0,)))
    ys1 = _trans_head(rows5, trg, trb, wtr, btr)

    g2 = _sort_pad(rows5, 5, 5, BUCKET)
    g2 = _gau_blocks(g2, 1, wts, None)
    onesb = jnp.full((1, BUCKET), 1.0 / BUCKET, jnp.float32)
    row1 = _dot(onesb, g2, ((1,), (0,)))
    ys2 = _trans_head(row1, trg, trb, wtr, btr)

    iota0 = jax.lax.broadcasted_iota(jnp.int32, (1, n_rows), 1)
    w0 = (iota0 < n1).astype(jnp.float32)
    sum0 = _dot(w0, ys0_ref[0], ((1,), (0,)))
    ones5 = jnp.ones((1, 5), jnp.float32)
    sum1 = _dot(ones5, ys1, ((1,), (0,)))
    y = (sum0 + sum1 + ys2) * (1.0 / (n1 + 5 + 1))
    m = jnp.mean(y, axis=1, keepdims=True)
    v = jnp.mean((y - m) ** 2, axis=1, keepdims=True)
    y = (y - m) / jnp.sqrt(v + 1e-5) * og_ref[...] + ob_ref[...]
    y = jnp.where(y >= 0, y, 0.01 * y)
    o_ref[0] = _dot(y, wout_ref[...], ((1,), (0,))) + bout_ref[...]


def _tail(gys, ys0, n1, wts, out_g, out_b, W_out, b_out):
    (lng, lnb, wuv, buv, wz, bz, gam, bet, wo, bo, trg, trb, wtr, btr) = wts
    Bn, n_rows, _ = gys.shape
    od = W_out.shape[1]
    const3 = lambda shp: pl.BlockSpec(shp, lambda s: (0, 0, 0))
    const2 = lambda shp: pl.BlockSpec(shp, lambda s: (0, 0))
    return pl.pallas_call(
        functools.partial(_tail_body, n1, None),
        grid=(Bn,),
        in_specs=[
            pl.BlockSpec((1, n_rows, D), lambda s: (s, 0, 0)),
            pl.BlockSpec((1, n_rows, D), lambda s: (s, 0, 0)),
            const2((N_BLOCK, D)), const2((N_BLOCK, D)),
            const3((N_BLOCK, D, 2 * E)), const2((N_BLOCK, 2 * E)),
            const3((N_BLOCK, D, S)), const2((N_BLOCK, S)),
            const2((2 * N_BLOCK, S)), const2((2 * N_BLOCK, S)),
            const3((N_BLOCK, E, D)), const2((N_BLOCK, D)),
            const2((1, D)), const2((1, D)),
            const2((D, D)), const2((1, D)),
            const2((1, D)), const2((1, D)),
            const2((D, od)), const2((1, od)),
        ],
        out_specs=pl.BlockSpec((1, 1, od), lambda s: (s, 0, 0)),
        out_shape=jax.ShapeDtypeStruct((Bn, 1, od), jnp.float32),
    )(gys, ys0, lng, lnb, wuv, buv, wz, bz, gam, bet, wo, bo, trg, trb,
      wtr, btr, out_g.reshape(1, D), out_b.reshape(1, D), W_out,
      b_out.reshape(1, od))[:, 0]


def _final_body(ys_ref, og_ref, ob_ref, wout_ref, bout_ref, o_ref):
    Bn = ys_ref.shape[0]
    rows = [jnp.mean(ys_ref[i], axis=0, keepdims=True) for i in range(Bn)]
    y = jnp.concatenate(rows, axis=0)
    m = jnp.mean(y, axis=1, keepdims=True)
    v = jnp.mean((y - m) ** 2, axis=1, keepdims=True)
    y = (y - m) / jnp.sqrt(v + 1e-5) * og_ref[...] + ob_ref[...]
    y = jnp.where(y >= 0, y, 0.01 * y)
    o_ref[...] = _dot(y, wout_ref[...], ((1,), (0,))) + bout_ref[...]


def _final(ys, out_g, out_b, W_out, b_out):
    Bn, nrows, _ = ys.shape
    od = W_out.shape[1]
    return pl.pallas_call(
        _final_body,
        in_specs=[
            pl.BlockSpec((Bn, nrows, D), lambda: (0, 0, 0)),
            pl.BlockSpec((1, D), lambda: (0, 0)),
            pl.BlockSpec((1, D), lambda: (0, 0)),
            pl.BlockSpec((D, od), lambda: (0, 0)),
            pl.BlockSpec((1, od), lambda: (0, 0)),
        ],
        out_specs=pl.BlockSpec((Bn, od), lambda: (0, 0)),
        out_shape=jax.ShapeDtypeStruct((Bn, od), jnp.float32),
    )(ys, out_g.reshape(1, D), out_b.reshape(1, D), W_out,
      b_out.reshape(1, od))


N0 = 16384
NP0 = 16896
CHUNK = 128
ROWS_PER_W = 2048


def _sc_sort_gather(x_flat, idx_flat, pad_blk):
    mesh = plsc.VectorSubcoreMesh(core_axis_name="c", subcore_axis_name="s")

    @functools.partial(
        pl.kernel, mesh=mesh,
        out_type=jax.ShapeDtypeStruct((4 * NP0, D), jnp.float32),
        scratch_types=[
            pltpu.VMEM((CHUNK,), jnp.int32),
            pltpu.VMEM((CHUNK, D), jnp.float32),
            pltpu.VMEM((CHUNK,), jnp.int32),
            pltpu.VMEM((CHUNK, D), jnp.float32),
            pltpu.SemaphoreType.DMA,
            pltpu.SemaphoreType.DMA,
        ],
    )
    def k(x_hbm, idx_hbm, pad_hbm, out_hbm, idx_a, rows_a, idx_b, rows_b,
          sem_a, sem_b):
        wid = lax.axis_index("s") * 2 + lax.axis_index("c")
        smp = wid // 8
        part = wid % 8
        src = smp * N0 + part * ROWS_PER_W
        dst = smp * NP0 + part * ROWS_PER_W
        idx_v = [idx_a, idx_b]
        rows_v = [rows_a, rows_b]
        sems = [sem_a, sem_b]
        nch = ROWS_PER_W // CHUNK
        cps = [None, None]
        for c in range(nch):
            b = c % 2
            pltpu.sync_copy(idx_hbm.at[pl.ds(src + c * CHUNK, CHUNK)],
                            idx_v[b])
            cps[b] = pltpu.async_copy(x_hbm.at[idx_v[b]], rows_v[b], sems[b])
            if c > 0:
                pb = (c - 1) % 2
                cps[pb].wait()
                pltpu.sync_copy(rows_v[pb],
                                out_hbm.at[pl.ds(dst + (c - 1) * CHUNK,
                                                 CHUNK)])
        cps[(nch - 1) % 2].wait()
        pltpu.sync_copy(rows_v[(nch - 1) % 2],
                        out_hbm.at[pl.ds(dst + (nch - 1) * CHUNK, CHUNK)])

        @pl.when(part == 0)
        def _():
            for c in range(4):
                pltpu.sync_copy(pad_hbm.at[pl.ds(c * CHUNK, CHUNK)], rows_a)
                pltpu.sync_copy(
                    rows_a,
                    out_hbm.at[pl.ds(smp * NP0 + N0 + c * CHUNK, CHUNK)])

    return k(x_flat, idx_flat, pad_blk)


def _pad_rows(n_pad):
    ids = np.arange(n_pad)
    pad = np.zeros((n_pad, D), np.float32)
    pad[ids, ids % D] = 1.0
    return jnp.asarray(pad)


def kernel(xs, W_in, b_in, blk_ln_g, blk_ln_b, blk_Wuv, blk_buv, blk_Wz,
           blk_bz, blk_gam, blk_bet, blk_Wo, blk_bo, tr_g, tr_b, W_tr, b_tr,
           out_g, out_b, W_out, b_out):
    Bn = xs.shape[0]
    wts = (blk_ln_g, blk_ln_b, blk_Wuv, blk_buv, blk_Wz, blk_bz,
           blk_gam.reshape(2 * N_BLOCK, S), blk_bet.reshape(2 * N_BLOCK, S),
           blk_Wo, blk_bo, tr_g.reshape(1, D), tr_b.reshape(1, D), W_tr,
           b_tr.reshape(1, D))

    x = _proj(xs, W_in, b_in)

    cw = _cosine_weights(x)
    order = jnp.argsort(-cw, axis=1).astype(jnp.int32)
    offs = (jnp.arange(Bn, dtype=jnp.int32) * N0)[:, None]
    idx_flat = (order + offs).reshape(-1)
    pad_blk = jnp.concatenate(
        [_pad_rows(BUCKET), jnp.zeros((512 - BUCKET, D), jnp.float32)], axis=0)
    g = _sc_sort_gather(x.reshape(Bn * N0, D), idx_flat,
                        pad_blk).reshape(Bn, NP0, D)
    gys, ys_l = _gau_level(g, 8, wts)

    n_bucket0 = (N0 + BUCKET) // BUCKET
    return _tail(gys, ys_l, n_bucket0, wts, out_g, out_b, W_out, b_out)


# speedup vs baseline: 1.1618x; 1.1618x over previous
import functools

import numpy as np
import jax
import jax.numpy as jnp
from jax import lax
from jax.experimental import pallas as pl
from jax.experimental.pallas import tpu as pltpu
from jax.experimental.pallas import tpu_sc as plsc

D = 128
E = 256
S = 64
BUCKET = 64
N_BLOCK = 2


def _dot(a, b, dims):
    return jax.lax.dot_general(a, b, (dims, ((), ())),
                               preferred_element_type=jnp.float32)


def _dot_fast(a, b, dims):
    return jax.lax.dot_general(a.astype(jnp.bfloat16), b.astype(jnp.bfloat16),
                               (dims, ((), ())),
                               preferred_element_type=jnp.float32)


def _proj_body(xs_ref, w_ref, b_ref, o_ref):
    o_ref[0] = _dot(xs_ref[0], w_ref[...], ((1,), (0,))) + b_ref[...]


def _proj(xs, W_in, b_in):
    Bn, n0, din = xs.shape
    RB = 2048
    return pl.pallas_call(
        _proj_body,
        grid=(Bn, n0 // RB),
        in_specs=[
            pl.BlockSpec((1, RB, din), lambda s, r: (s, r, 0)),
            pl.BlockSpec((din, D), lambda s, r: (0, 0)),
            pl.BlockSpec((1, D), lambda s, r: (0, 0)),
        ],
        out_specs=pl.BlockSpec((1, RB, D), lambda s, r: (s, r, 0)),
        out_shape=jax.ShapeDtypeStruct((Bn, n0, D), jnp.float32),
    )(xs, W_in, b_in.reshape(1, D))


def _cw_body(x_ref, cw_ref):
    x = x_ref[0]
    n = x.shape[0]
    ones = jnp.ones((1, D), jnp.float32)
    lens2 = _dot(ones, x * x, ((1,), (1,)))
    m = jnp.max(lens2)
    iota = jax.lax.broadcasted_iota(jnp.int32, (1, n), 1)
    idx = jnp.min(jnp.where(lens2 == m, iota, n))
    onehot = (iota == idx).astype(jnp.float32)
    v1 = _dot(onehot, x, ((1,), (0,)))
    v1n = jnp.sqrt(jnp.sum(v1 * v1))
    dots = _dot(v1, x, ((1,), (1,)))
    denom = jnp.maximum(v1n * jnp.sqrt(lens2), 1e-8)
    cw_ref[0] = dots / denom


def _cosine_weights(x):
    Bn, n, _ = x.shape
    out = pl.pallas_call(
        _cw_body,
        grid=(Bn,),
        in_specs=[pl.BlockSpec((1, n, D), lambda s: (s, 0, 0))],
        out_specs=pl.BlockSpec((1, 1, n), lambda s: (s, 0, 0)),
        out_shape=jax.ShapeDtypeStruct((Bn, 1, n), jnp.float32),
    )(x)
    return out[:, 0]


def _gau_body(G, g_ref, lng_ref, lnb_ref, wuv_ref, buv_ref, wz_ref, bz_ref,
              gam_ref, bet_ref, wo_ref, bo_ref, trg_ref, trb_ref, wtr_ref,
              btr_ref, gys_ref, ys_ref):
    x = g_ref[0]
    PAIR = G % 2 == 0
    if PAIR:
        r2 = jax.lax.broadcasted_iota(jnp.int32, (2 * BUCKET, 2 * BUCKET), 0)
        c2 = jax.lax.broadcasted_iota(jnp.int32, (2 * BUCKET, 2 * BUCKET), 1)
        bdmask = (r2 // BUCKET == c2 // BUCKET).astype(jnp.float32)
    for j in range(N_BLOCK):
        mean = jnp.mean(x, axis=1, keepdims=True)
        var = jnp.mean((x - mean) ** 2, axis=1, keepdims=True)
        xn = (x - mean) / jnp.sqrt(var + 1e-5) * lng_ref[j:j + 1] \
            + lnb_ref[j:j + 1]
        uv = _dot(xn.astype(jnp.bfloat16), wuv_ref[j], ((1,), (0,))) \
            + buv_ref[j:j + 1]
        uv = uv * jax.nn.sigmoid(uv)
        u = uv[:, :E]
        v = uv[:, E:]
        z = _dot(xn, wz_ref[j], ((1,), (0,))) + bz_ref[j:j + 1]
        q = z * gam_ref[2 * j:2 * j + 1] + bet_ref[2 * j:2 * j + 1]
        k = z * gam_ref[2 * j + 1:2 * j + 2] + bet_ref[2 * j + 1:2 * j + 2]
        outs = []
        if PAIR:
            for t in range(G // 2):
                sl = slice(t * 2 * BUCKET, (t + 1) * 2 * BUCKET)
                qk = _dot(q[sl], k[sl], ((1,), (1,))) * (1.0 / BUCKET)
                A = jnp.square(jnp.maximum(qk, 0.0)) * bdmask
                outs.append(_dot(A, v[sl], ((1,), (0,))))
        else:
            for t in range(G):
                sl = slice(t * BUCKET, (t + 1) * BUCKET)
                qk = _dot(q[sl], k[sl], ((1,), (1,))) * (1.0 / BUCKET)
                A = jnp.square(jnp.maximum(qk, 0.0))
                outs.append(_dot(A, v[sl], ((1,), (0,))))
        av = jnp.concatenate(outs, axis=0) if len(outs) > 1 else outs[0]
        x = x + _dot((u * av).astype(jnp.bfloat16), wo_ref[j], ((1,), (0,))) \
            + bo_ref[j:j + 1]
    rg = jax.lax.broadcasted_iota(jnp.int32, (G, G * BUCKET), 0)
    cg = jax.lax.broadcasted_iota(jnp.int32, (G, G * BUCKET), 1)
    pool = jnp.where(cg // BUCKET == rg, 1.0 / BUCKET, 0.0)
    rows = _dot(pool, x, ((1,), (0,)))
    gys_ref[0] = rows
    m2 = jnp.mean(rows, axis=1, keepdims=True)
    v2 = jnp.mean((rows - m2) ** 2, axis=1, keepdims=True)
    t_ = (rows - m2) / jnp.sqrt(v2 + 1e-5) * trg_ref[...] + trb_ref[...]
    t_ = jnp.where(t_ >= 0, t_, 0.01 * t_)
    ys_ref[0] = _dot(t_, wtr_ref[...], ((1,), (0,))) + btr_ref[...]


def _gau_level(g, G, wts):
    (lng, lnb, wuv, buv, wz, bz, gam, bet, wo, bo, trg, trb, wtr, btr) = wts
    Bn, rows, _ = g.shape
    nbp = rows // BUCKET
    grid = (Bn, nbp // G)
    const3 = lambda shp: pl.BlockSpec(shp, lambda s, b: (0, 0, 0))
    const2 = lambda shp: pl.BlockSpec(shp, lambda s, b: (0, 0))
    return pl.pallas_call(
        functools.partial(_gau_body, G),
        grid=grid,
        in_specs=[
            pl.BlockSpec((1, G * BUCKET, D), lambda s, b: (s, b, 0)),
            const2((N_BLOCK, D)), const2((N_BLOCK, D)),
            const3((N_BLOCK, D, 2 * E)), const2((N_BLOCK, 2 * E)),
            const3((N_BLOCK, D, S)), const2((N_BLOCK, S)),
            const2((2 * N_BLOCK, S)), const2((2 * N_BLOCK, S)),
            const3((N_BLOCK, E, D)), const2((N_BLOCK, D)),
            const2((1, D)), const2((1, D)),
            const2((D, D)), const2((1, D)),
        ],
        out_specs=[
            pl.BlockSpec((1, G, D), lambda s, b: (s, b, 0)),
            pl.BlockSpec((1, G, D), lambda s, b: (s, b, 0)),
        ],
        out_shape=[
            jax.ShapeDtypeStruct((Bn, nbp, D), jnp.float32),
            jax.ShapeDtypeStruct((Bn, nbp, D), jnp.float32),
        ],
    )(g, lng, lnb, wuv, buv, wz, bz, gam, bet, wo, bo, trg, trb, wtr, btr)


def _rank_desc(cw_row, valid_row, n):
    ii = jax.lax.broadcasted_iota(jnp.int32, (n, n), 0)
    jj = jax.lax.broadcasted_iota(jnp.int32, (n, n), 1)
    eye = (ii == jj).astype(jnp.float32)
    cw_col = _dot(eye, cw_row, ((1,), (1,)))
    gt = jnp.where((cw_row > cw_col) & valid_row, 1.0, 0.0)
    tie = jnp.where((cw_row == cw_col) & (jj < ii) & valid_row, 1.0, 0.0)
    ones = jnp.ones((n, 1), jnp.float32)
    return _dot(gt + tie, ones, ((1,), (0,)))


def _sort_pad(x, n_valid, n_rows, n_out):
    iota_r = jax.lax.broadcasted_iota(jnp.int32, (1, n_rows), 1)
    valid = iota_r < n_valid
    ones = jnp.ones((1, D), jnp.float32)
    lens2 = _dot(ones, x * x, ((1,), (1,)))
    lens2m = jnp.where(valid, lens2, -1.0)
    m = jnp.max(lens2m)
    idx = jnp.min(jnp.where(lens2m == m, iota_r, n_rows))
    onehot = (iota_r == idx).astype(jnp.float32)
    v1 = _dot(onehot, x, ((1,), (0,)))
    v1n = jnp.sqrt(jnp.sum(v1 * v1))
    dots = _dot(v1, x, ((1,), (1,)))
    denom = jnp.maximum(v1n * jnp.sqrt(jnp.maximum(lens2, 0.0)), 1e-8)
    cw = jnp.where(valid, dots / denom, -3.0)
    rank = _rank_desc(cw, valid, n_rows)
    validc = jax.lax.broadcasted_iota(jnp.int32, (n_rows, 1), 0) < n_valid
    ranks = jnp.where(validc, rank, -1.0)
    kk = jax.lax.broadcasted_iota(jnp.int32, (n_rows, n_out), 1)
    PT = jnp.where(ranks == kk.astype(jnp.float32), 1.0, 0.0)
    xs_sorted = _dot(PT, x, ((0,), (0,)))
    ko = jax.lax.broadcasted_iota(jnp.int32, (n_out, D), 0)
    do = jax.lax.broadcasted_iota(jnp.int32, (n_out, D), 1)
    padfill = jnp.where((ko >= n_valid) & (do == (ko - n_valid) % D),
                        1.0, 0.0)
    return xs_sorted + padfill


def _gau_blocks(x, G, wts, bdmask):
    (lng, lnb, wuv, buv, wz, bz, gam, bet, wo, bo) = wts
    for j in range(N_BLOCK):
        mean = jnp.mean(x, axis=1, keepdims=True)
        var = jnp.mean((x - mean) ** 2, axis=1, keepdims=True)
        xn = (x - mean) / jnp.sqrt(var + 1e-5) * lng[j:j + 1] + lnb[j:j + 1]
        uv = _dot(xn.astype(jnp.bfloat16), wuv[j], ((1,), (0,))) + buv[j:j + 1]
        uv = uv * jax.nn.sigmoid(uv)
        u = uv[:, :E]
        v = uv[:, E:]
        z = _dot(xn, wz[j], ((1,), (0,))) + bz[j:j + 1]
        q = z * gam[2 * j:2 * j + 1] + bet[2 * j:2 * j + 1]
        k = z * gam[2 * j + 1:2 * j + 2] + bet[2 * j + 1:2 * j + 2]
        qk = _dot(q, k, ((1,), (1,))) * (1.0 / BUCKET)
        A = jnp.square(jnp.maximum(qk, 0.0))
        if bdmask is not None:
            A = A * bdmask
        av = _dot(A, v, ((1,), (0,)))
        x = x + _dot((u * av).astype(jnp.bfloat16), wo[j], ((1,), (0,))) \
            + bo[j:j + 1]
    return x


def _trans_head(rows, trg, trb, wtr, btr):
    m = jnp.mean(rows, axis=1, keepdims=True)
    v = jnp.mean((rows - m) ** 2, axis=1, keepdims=True)
    t_ = (rows - m) / jnp.sqrt(v + 1e-5) * trg + trb
    t_ = jnp.where(t_ >= 0, t_, 0.01 * t_)
    return _dot(t_, wtr, ((1,), (0,))) + btr


def _tail_body(n1, gw, g_ref, ys0_ref, lng_ref, lnb_ref, wuv_ref, buv_ref,
               wz_ref, bz_ref, gam_ref, bet_ref, wo_ref, bo_ref, trg_ref,
               trb_ref, wtr_ref, btr_ref, og_ref, ob_ref, wout_ref, bout_ref,
               o_ref):
    x = g_ref[0]
    n_rows = x.shape[0]
    wts = (lng_ref[...], lnb_ref[...], wuv_ref[...], buv_ref[...],
           wz_ref[...], bz_ref[...], gam_ref[...], bet_ref[...],
           wo_ref[...], bo_ref[...])
    trg, trb = trg_ref[...], trb_ref[...]
    wtr, btr = wtr_ref[...], btr_ref[...]

    g1 = _sort_pad(x, n1, n_rows, 5 * BUCKET)
    r5 = jax.lax.broadcasted_iota(jnp.int32, (5 * BUCKET, 5 * BUCKET), 0)
    c5 = jax.lax.broadcasted_iota(jnp.int32, (5 * BUCKET, 5 * BUCKET), 1)
    bdmask = (r5 // BUCKET == c5 // BUCKET).astype(jnp.float32)
    g1 = _gau_blocks(g1, 5, wts, bdmask)
    rg = jax.lax.broadcasted_iota(jnp.int32, (5, 5 * BUCKET), 0)
    cg = jax.lax.broadcasted_iota(jnp.int32, (5, 5 * BUCKET), 1)
    pool1 = jnp.where(cg // BUCKET == rg, 1.0 / BUCKET, 0.0)
    rows5 = _dot(pool1, g1, ((1,), (0,)))
    ys1 = _trans_head(rows5, trg, trb, wtr, btr)

    g2 = _sort_pad(rows5, 5, 5, BUCKET)
    g2 = _gau_blocks(g2, 1, wts, None)
    onesb = jnp.full((1, BUCKET), 1.0 / BUCKET, jnp.float32)
    row1 = _dot(onesb, g2, ((1,), (0,)))
    ys2 = _trans_head(row1, trg, trb, wtr, btr)

    iota0 = jax.lax.broadcasted_iota(jnp.int32, (1, n_rows), 1)
    w0 = (iota0 < n1).astype(jnp.float32)
    sum0 = _dot(w0, ys0_ref[0], ((1,), (0,)))
    ones5 = jnp.ones((1, 5), jnp.float32)
    sum1 = _dot(ones5, ys1, ((1,), (0,)))
    y = (sum0 + sum1 + ys2) * (1.0 / (n1 + 5 + 1))
    m = jnp.mean(y, axis=1, keepdims=True)
    v = jnp.mean((y - m) ** 2, axis=1, keepdims=True)
    y = (y - m) / jnp.sqrt(v + 1e-5) * og_ref[...] + ob_ref[...]
    y = jnp.where(y >= 0, y, 0.01 * y)
    o_ref[0] = _dot(y, wout_ref[...], ((1,), (0,))) + bout_ref[...]


def _tail(gys, ys0, n1, wts, out_g, out_b, W_out, b_out):
    (lng, lnb, wuv, buv, wz, bz, gam, bet, wo, bo, trg, trb, wtr, btr) = wts
    Bn, n_rows, _ = gys.shape
    od = W_out.shape[1]
    const3 = lambda shp: pl.BlockSpec(shp, lambda s: (0, 0, 0))
    const2 = lambda shp: pl.BlockSpec(shp, lambda s: (0, 0))
    return pl.pallas_call(
        functools.partial(_tail_body, n1, None),
        grid=(Bn,),
        in_specs=[
            pl.BlockSpec((1, n_rows, D), lambda s: (s, 0, 0)),
            pl.BlockSpec((1, n_rows, D), lambda s: (s, 0, 0)),
            const2((N_BLOCK, D)), const2((N_BLOCK, D)),
            const3((N_BLOCK, D, 2 * E)), const2((N_BLOCK, 2 * E)),
            const3((N_BLOCK, D, S)), const2((N_BLOCK, S)),
            const2((2 * N_BLOCK, S)), const2((2 * N_BLOCK, S)),
            const3((N_BLOCK, E, D)), const2((N_BLOCK, D)),
            const2((1, D)), const2((1, D)),
            const2((D, D)), const2((1, D)),
            const2((1, D)), const2((1, D)),
            const2((D, od)), const2((1, od)),
        ],
        out_specs=pl.BlockSpec((1, 1, od), lambda s: (s, 0, 0)),
        out_shape=jax.ShapeDtypeStruct((Bn, 1, od), jnp.float32),
    )(gys, ys0, lng, lnb, wuv, buv, wz, bz, gam, bet, wo, bo, trg, trb,
      wtr, btr, out_g.reshape(1, D), out_b.reshape(1, D), W_out,
      b_out.reshape(1, od))[:, 0]


def _final_body(ys_ref, og_ref, ob_ref, wout_ref, bout_ref, o_ref):
    Bn = ys_ref.shape[0]
    rows = [jnp.mean(ys_ref[i], axis=0, keepdims=True) for i in range(Bn)]
    y = jnp.concatenate(rows, axis=0)
    m = jnp.mean(y, axis=1, keepdims=True)
    v = jnp.mean((y - m) ** 2, axis=1, keepdims=True)
    y = (y - m) / jnp.sqrt(v + 1e-5) * og_ref[...] + ob_ref[...]
    y = jnp.where(y >= 0, y, 0.01 * y)
    o_ref[...] = _dot(y, wout_ref[...], ((1,), (0,))) + bout_ref[...]


def _final(ys, out_g, out_b, W_out, b_out):
    Bn, nrows, _ = ys.shape
    od = W_out.shape[1]
    return pl.pallas_call(
        _final_body,
        in_specs=[
            pl.BlockSpec((Bn, nrows, D), lambda: (0, 0, 0)),
            pl.BlockSpec((1, D), lambda: (0, 0)),
            pl.BlockSpec((1, D), lambda: (0, 0)),
            pl.BlockSpec((D, od), lambda: (0, 0)),
            pl.BlockSpec((1, od), lambda: (0, 0)),
        ],
        out_specs=pl.BlockSpec((Bn, od), lambda: (0, 0)),
        out_shape=jax.ShapeDtypeStruct((Bn, od), jnp.float32),
    )(ys, out_g.reshape(1, D), out_b.reshape(1, D), W_out,
      b_out.reshape(1, od))


N0 = 16384
NP0 = 17408
CHUNK = 128
ROWS_PER_W = 2048


def _sc_sort_gather(x_flat, idx_flat, pad_blk):
    mesh = plsc.VectorSubcoreMesh(core_axis_name="c", subcore_axis_name="s")

    @functools.partial(
        pl.kernel, mesh=mesh,
        out_type=jax.ShapeDtypeStruct((4 * NP0, D), jnp.float32),
        scratch_types=[
            pltpu.VMEM((CHUNK,), jnp.int32),
            pltpu.VMEM((CHUNK, D), jnp.float32),
            pltpu.VMEM((CHUNK,), jnp.int32),
            pltpu.VMEM((CHUNK, D), jnp.float32),
            pltpu.SemaphoreType.DMA,
            pltpu.SemaphoreType.DMA,
        ],
    )
    def k(x_hbm, idx_hbm, pad_hbm, out_hbm, idx_a, rows_a, idx_b, rows_b,
          sem_a, sem_b):
        wid = lax.axis_index("s") * 2 + lax.axis_index("c")
        smp = wid // 8
        part = wid % 8
        src = smp * N0 + part * ROWS_PER_W
        dst = smp * NP0 + part * ROWS_PER_W
        idx_v = [idx_a, idx_b]
        rows_v = [rows_a, rows_b]
        sems = [sem_a, sem_b]
        nch = ROWS_PER_W // CHUNK
        cps = [None, None]
        for c in range(nch):
            b = c % 2
            pltpu.sync_copy(idx_hbm.at[pl.ds(src + c * CHUNK, CHUNK)],
                            idx_v[b])
            cps[b] = pltpu.async_copy(x_hbm.at[idx_v[b]], rows_v[b], sems[b])
            if c > 0:
                pb = (c - 1) % 2
                cps[pb].wait()
                pltpu.sync_copy(rows_v[pb],
                                out_hbm.at[pl.ds(dst + (c - 1) * CHUNK,
                                                 CHUNK)])
        cps[(nch - 1) % 2].wait()
        pltpu.sync_copy(rows_v[(nch - 1) % 2],
                        out_hbm.at[pl.ds(dst + (nch - 1) * CHUNK, CHUNK)])

        @pl.when(part == 0)
        def _():
            for c in range((NP0 - N0) // CHUNK):
                pltpu.sync_copy(pad_hbm.at[pl.ds(c * CHUNK, CHUNK)], rows_a)
                pltpu.sync_copy(
                    rows_a,
                    out_hbm.at[pl.ds(smp * NP0 + N0 + c * CHUNK, CHUNK)])

    return k(x_flat, idx_flat, pad_blk)


def _pad_rows(n_pad):
    ids = np.arange(n_pad)
    pad = np.zeros((n_pad, D), np.float32)
    pad[ids, ids % D] = 1.0
    return jnp.asarray(pad)


def kernel(xs, W_in, b_in, blk_ln_g, blk_ln_b, blk_Wuv, blk_buv, blk_Wz,
           blk_bz, blk_gam, blk_bet, blk_Wo, blk_bo, tr_g, tr_b, W_tr, b_tr,
           out_g, out_b, W_out, b_out):
    Bn = xs.shape[0]
    wts = (blk_ln_g, blk_ln_b, blk_Wuv, blk_buv, blk_Wz, blk_bz,
           blk_gam.reshape(2 * N_BLOCK, S), blk_bet.reshape(2 * N_BLOCK, S),
           blk_Wo, blk_bo, tr_g.reshape(1, D), tr_b.reshape(1, D), W_tr,
           b_tr.reshape(1, D))

    x = _proj(xs, W_in, b_in)

    cw = _cosine_weights(x)
    order = jnp.argsort(-cw, axis=1).astype(jnp.int32)
    offs = (jnp.arange(Bn, dtype=jnp.int32) * N0)[:, None]
    idx_flat = (order + offs).reshape(-1)
    pad_blk = jnp.concatenate(
        [_pad_rows(BUCKET), jnp.zeros((NP0 - N0 - BUCKET, D), jnp.float32)],
        axis=0)
    g = _sc_sort_gather(x.reshape(Bn * N0, D), idx_flat,
                        pad_blk).reshape(Bn, NP0, D)
    gys, ys_l = _gau_level(g, 16, wts)

    n_bucket0 = (N0 + BUCKET) // BUCKET
    return _tail(gys, ys_l, n_bucket0, wts, out_g, out_b, W_out, b_out)
